# baseline (speedup 1.0000x reference)
import functools

import jax
import jax.numpy as jnp
from jax import lax
from jax.experimental import pallas as pl
from jax.experimental.pallas import tpu as pltpu
from jax.experimental.pallas import tpu_sc as plsc

BATCH = 16384
CTX = 6
EMBED_DIM = 128
OUTPUTS = 91

NC = 2
NS = 16
NW = NC * NS

N_IDX = BATCH * CTX
PER_W = N_IDX // NW
CHUNK = 128
ITERS = PER_W // CHUNK
NBUF = 4


def _sc_gather_body(iters, nbuf, idx_hbm, table_hbm, out_hbm, idx_v, bufs, gsem, ssem):
    per_w = iters * CHUNK
    c = lax.axis_index("c")
    s = lax.axis_index("s")
    wid = s * NC + c
    base = wid * per_w
    pltpu.sync_copy(idx_hbm.at[wid], idx_v)

    def start_gather(b, j):
        pltpu.async_copy(table_hbm.at[idx_v.at[j]], bufs.at[b], gsem.at[b])

    def start_store(b, j):
        pltpu.async_copy(
            bufs.at[b], out_hbm.at[pl.ds(base + j * CHUNK, CHUNK)], ssem.at[b]
        )

    def wait_gather(b):
        pltpu.make_async_copy(
            table_hbm.at[idx_v.at[0]], bufs.at[b], gsem.at[b]
        ).wait()

    def wait_store(b):
        pltpu.make_async_copy(
            bufs.at[b], out_hbm.at[pl.ds(base, CHUNK)], ssem.at[b]
        ).wait()

    for b in range(nbuf):
        start_gather(b, b)

    def outer(t, carry):
        j0 = t * nbuf
        for b in range(nbuf):
            wait_gather(b)
            start_store(b, j0 + b)
        for b in range(nbuf):
            wait_store(b)
            start_gather(b, j0 + nbuf + b)
        return carry

    lax.fori_loop(0, iters // nbuf - 1, outer, 0)

    j0 = iters - nbuf
    for b in range(nbuf):
        wait_gather(b)
        start_store(b, j0 + b)
    for b in range(nbuf):
        wait_store(b)


@functools.partial(jax.jit, static_argnames=("nbuf",))
def _sc_gather(idx3, table, nbuf=NBUF):
    nw, iters, chunk = idx3.shape
    mesh = plsc.VectorSubcoreMesh(core_axis_name="c", subcore_axis_name="s")
    return pl.kernel(
        functools.partial(_sc_gather_body, iters, nbuf),
        mesh=mesh,
        out_type=jax.ShapeDtypeStruct((nw * iters * chunk, EMBED_DIM), jnp.float32),
        scratch_types=[
            pltpu.VMEM((iters, CHUNK), jnp.int32),
            pltpu.VMEM((nbuf, CHUNK, EMBED_DIM), jnp.float32),
            pltpu.SemaphoreType.DMA((nbuf,)),
            pltpu.SemaphoreType.DMA((nbuf,)),
        ],
    )(idx3, table)


def _mlp_body(*refs):
    x_refs = refs[:CTX]
    w1_ref, b1_ref, w2_ref, b2_ref, o_ref = refs[CTX:]
    h = jnp.dot(x_refs[0][...], w1_ref[0], preferred_element_type=jnp.float32)
    for k in range(1, CTX):
        h += jnp.dot(x_refs[k][...], w1_ref[k], preferred_element_type=jnp.float32)
    h = jnp.maximum(h + b1_ref[...], 0.0)
    o_ref[...] = (
        jnp.dot(h, w2_ref[...], preferred_element_type=jnp.float32) + b2_ref[...]
    )


def _x_spec(k, bm, bp):
    nb = bp // bm
    return pl.BlockSpec((bm, EMBED_DIM), lambda i, _k=k, _nb=nb: (_k * _nb + i, 0))


@functools.partial(jax.jit, static_argnames=("bm",))
def _mlp(xkm, w1k, b1, w2, b2, bm=1024):
    bp = xkm.shape[0] // CTX
    nb = bp // bm
    return pl.pallas_call(
        _mlp_body,
        grid=(nb,),
        in_specs=[_x_spec(k, bm, bp) for k in range(CTX)]
        + [
            pl.BlockSpec((CTX, EMBED_DIM, EMBED_DIM), lambda i: (0, 0, 0)),
            pl.BlockSpec((1, EMBED_DIM), lambda i: (0, 0)),
            pl.BlockSpec((EMBED_DIM, OUTPUTS), lambda i: (0, 0)),
            pl.BlockSpec((1, OUTPUTS), lambda i: (0, 0)),
        ],
        out_specs=pl.BlockSpec((bm, OUTPUTS), lambda i: (i, 0)),
        out_shape=jax.ShapeDtypeStruct((bp, OUTPUTS), jnp.float32),
    )(*([xkm] * CTX), w1k, b1, w2, b2)


NPARTS = 2


def kernel(inputs, emb_table, W1, b1, W2, b2):
    bp = BATCH // NPARTS
    iters_p = ITERS // NPARTS
    nbuf = min(NBUF, iters_p)
    idx_t = inputs.astype(jnp.int32).T
    w1k = W1.reshape(CTX, EMBED_DIM, EMBED_DIM)
    b1r = b1.reshape(1, EMBED_DIM)
    b2r = b2.reshape(1, OUTPUTS)
    outs = []
    for p in range(NPARTS):
        idx3 = idx_t[:, p * bp:(p + 1) * bp].reshape(NW, iters_p, CHUNK)
        xkm = _sc_gather(idx3, emb_table, nbuf=nbuf)
        outs.append(_mlp(xkm, w1k, b1r, b2=b2r, w2=W2))
    return jnp.concatenate(outs, axis=0)


# speedup vs baseline: 1.0310x; 1.0310x over previous
import functools

import jax
import jax.numpy as jnp
from jax import lax
from jax.experimental import pallas as pl
from jax.experimental.pallas import tpu as pltpu
from jax.experimental.pallas import tpu_sc as plsc

BATCH = 16384
CTX = 6
EMBED_DIM = 128
OUTPUTS = 91

NC = 2
NS = 16
NW = NC * NS

N_IDX = BATCH * CTX
PER_W = N_IDX // NW
CHUNK = 128
ITERS = PER_W // CHUNK
NBUF = 4


def _sc_gather_body(iters, nbuf, idx_hbm, table_hbm, out_hbm, idx_v, bufs, gsem, ssem):
    per_w = iters * CHUNK
    c = lax.axis_index("c")
    s = lax.axis_index("s")
    wid = s * NC + c
    base = wid * per_w
    pltpu.sync_copy(idx_hbm.at[wid], idx_v)

    def start_gather(b, j):
        pltpu.async_copy(table_hbm.at[idx_v.at[j]], bufs.at[b], gsem.at[b])

    def start_store(b, j):
        pltpu.async_copy(
            bufs.at[b], out_hbm.at[pl.ds(base + j * CHUNK, CHUNK)], ssem.at[b]
        )

    def wait_gather(b):
        pltpu.make_async_copy(
            table_hbm.at[idx_v.at[0]], bufs.at[b], gsem.at[b]
        ).wait()

    def wait_store(b):
        pltpu.make_async_copy(
            bufs.at[b], out_hbm.at[pl.ds(base, CHUNK)], ssem.at[b]
        ).wait()

    for b in range(nbuf):
        start_gather(b, b)

    def outer(t, carry):
        j0 = t * nbuf
        for b in range(nbuf):
            wait_gather(b)
            start_store(b, j0 + b)
        for b in range(nbuf):
            wait_store(b)
            start_gather(b, j0 + nbuf + b)
        return carry

    lax.fori_loop(0, iters // nbuf - 1, outer, 0)

    j0 = iters - nbuf
    for b in range(nbuf):
        wait_gather(b)
        start_store(b, j0 + b)
    for b in range(nbuf):
        wait_store(b)


@functools.partial(jax.jit, static_argnames=("nbuf",))
def _sc_gather(idx3, table, nbuf=NBUF):
    nw, iters, chunk = idx3.shape
    mesh = plsc.VectorSubcoreMesh(core_axis_name="c", subcore_axis_name="s")
    return pl.kernel(
        functools.partial(_sc_gather_body, iters, nbuf),
        mesh=mesh,
        out_type=jax.ShapeDtypeStruct((nw * iters * chunk, EMBED_DIM), jnp.float32),
        scratch_types=[
            pltpu.VMEM((iters, CHUNK), jnp.int32),
            pltpu.VMEM((nbuf, CHUNK, EMBED_DIM), jnp.float32),
            pltpu.SemaphoreType.DMA((nbuf,)),
            pltpu.SemaphoreType.DMA((nbuf,)),
        ],
    )(idx3, table)


def _mlp_body(*refs):
    x_refs = refs[:CTX]
    w1_ref, b1_ref, w2_ref, b2_ref, _base_ref, o_ref = refs[CTX:]
    h = jnp.dot(x_refs[0][...], w1_ref[0], preferred_element_type=jnp.float32)
    for k in range(1, CTX):
        h += jnp.dot(x_refs[k][...], w1_ref[k], preferred_element_type=jnp.float32)
    h = jnp.maximum(h + b1_ref[...], 0.0)
    o_ref[...] = (
        jnp.dot(h, w2_ref[...], preferred_element_type=jnp.float32) + b2_ref[...]
    )


def _x_spec(k, bm, bp):
    nb = bp // bm
    return pl.BlockSpec((bm, EMBED_DIM), lambda i, _k=k, _nb=nb: (_k * _nb + i, 0))


@functools.partial(jax.jit, static_argnames=("bm", "part", "donate_base"))
def _mlp_part(xkm, w1k, b1, w2, b2, base, part, bm=1024, donate_base=True):
    bp = xkm.shape[0] // CTX
    nb = bp // bm
    off = part * nb
    return pl.pallas_call(
        _mlp_body,
        grid=(nb,),
        in_specs=[_x_spec(k, bm, bp) for k in range(CTX)]
        + [
            pl.BlockSpec((CTX, EMBED_DIM, EMBED_DIM), lambda i: (0, 0, 0)),
            pl.BlockSpec((1, EMBED_DIM), lambda i: (0, 0)),
            pl.BlockSpec((EMBED_DIM, OUTPUTS), lambda i: (0, 0)),
            pl.BlockSpec((1, OUTPUTS), lambda i: (0, 0)),
            pl.BlockSpec(memory_space=pl.ANY),
        ],
        out_specs=pl.BlockSpec((bm, OUTPUTS), lambda i, _o=off: (_o + i, 0)),
        out_shape=jax.ShapeDtypeStruct((BATCH, OUTPUTS), jnp.float32),
        input_output_aliases={CTX + 4: 0},
    )(*([xkm] * CTX), w1k, b1, w2, b2, base)


NPARTS = 4


def kernel(inputs, emb_table, W1, b1, W2, b2):
    bp = BATCH // NPARTS
    iters_p = ITERS // NPARTS
    nbuf = min(NBUF, iters_p)
    while iters_p % nbuf:
        nbuf -= 1
    idx_t = inputs.astype(jnp.int32).T
    w1k = W1.reshape(CTX, EMBED_DIM, EMBED_DIM)
    b1r = b1.reshape(1, EMBED_DIM)
    b2r = b2.reshape(1, OUTPUTS)
    out = jnp.zeros((BATCH, OUTPUTS), jnp.float32)
    for p in range(NPARTS):
        idx3 = idx_t[:, p * bp:(p + 1) * bp].reshape(NW, iters_p, CHUNK)
        xkm = _sc_gather(idx3, emb_table, nbuf=nbuf)
        out = _mlp_part(xkm, w1k, b1r, W2, b2r, out, part=p)
    return out


# speedup vs baseline: 1.0836x; 1.0510x over previous
import functools

import jax
import jax.numpy as jnp
from jax import lax
from jax.experimental import pallas as pl
from jax.experimental.pallas import tpu as pltpu
from jax.experimental.pallas import tpu_sc as plsc

BATCH = 16384
CTX = 6
EMBED_DIM = 128
OUTPUTS = 91

NC = 2
NS = 16
NW = NC * NS

N_IDX = BATCH * CTX
PER_W = N_IDX // NW
CHUNK = 128
ITERS = PER_W // CHUNK
NBUF = 4


def _sc_gather_body(iters, nbuf, idx_hbm, table_hbm, out_hbm, idx_v, bufs, gsem, ssem):
    per_w = iters * CHUNK
    c = lax.axis_index("c")
    s = lax.axis_index("s")
    wid = s * NC + c
    base = wid * per_w
    pltpu.sync_copy(idx_hbm.at[wid], idx_v)

    def start_gather(b, j):
        pltpu.async_copy(table_hbm.at[idx_v.at[j]], bufs.at[b], gsem.at[b])

    def start_store(b, j):
        pltpu.async_copy(
            bufs.at[b], out_hbm.at[pl.ds(base + j * CHUNK, CHUNK)], ssem.at[b]
        )

    def wait_gather(b):
        pltpu.make_async_copy(
            table_hbm.at[idx_v.at[0]], bufs.at[b], gsem.at[b]
        ).wait()

    def wait_store(b):
        pltpu.make_async_copy(
            bufs.at[b], out_hbm.at[pl.ds(base, CHUNK)], ssem.at[b]
        ).wait()

    for b in range(nbuf):
        start_gather(b, b)

    def outer(t, carry):
        j0 = t * nbuf
        for b in range(nbuf):
            wait_gather(b)
            start_store(b, j0 + b)
        for b in range(nbuf):
            wait_store(b)
            start_gather(b, j0 + nbuf + b)
        return carry

    lax.fori_loop(0, iters // nbuf - 1, outer, 0)

    j0 = iters - nbuf
    for b in range(nbuf):
        wait_gather(b)
        start_store(b, j0 + b)
    for b in range(nbuf):
        wait_store(b)


@functools.partial(jax.jit, static_argnames=("nbuf",))
def _sc_gather(idx3, table, nbuf=NBUF):
    nw, iters, chunk = idx3.shape
    mesh = plsc.VectorSubcoreMesh(core_axis_name="c", subcore_axis_name="s")
    return pl.kernel(
        functools.partial(_sc_gather_body, iters, nbuf),
        mesh=mesh,
        out_type=jax.ShapeDtypeStruct((nw * iters * chunk, EMBED_DIM), jnp.float32),
        scratch_types=[
            pltpu.VMEM((iters, CHUNK), jnp.int32),
            pltpu.VMEM((nbuf, CHUNK, EMBED_DIM), jnp.float32),
            pltpu.SemaphoreType.DMA((nbuf,)),
            pltpu.SemaphoreType.DMA((nbuf,)),
        ],
    )(idx3, table)


def _mlp_compute(x_refs, w1_ref, b1_ref, w2_ref, b2_ref, o_ref):
    h = jnp.dot(x_refs[0][...], w1_ref[0], preferred_element_type=jnp.float32)
    for k in range(1, CTX):
        h += jnp.dot(x_refs[k][...], w1_ref[k], preferred_element_type=jnp.float32)
    h = jnp.maximum(h + b1_ref[...], 0.0)
    o_ref[...] = (
        jnp.dot(h, w2_ref[...], preferred_element_type=jnp.float32) + b2_ref[...]
    )


def _mlp_body(*refs):
    _mlp_compute(refs[:CTX], refs[CTX], refs[CTX + 1], refs[CTX + 2],
                 refs[CTX + 3], refs[CTX + 5])


def _mlp_body_nobase(*refs):
    _mlp_compute(refs[:CTX], refs[CTX], refs[CTX + 1], refs[CTX + 2],
                 refs[CTX + 3], refs[CTX + 4])


def _x_spec(k, bm, bp):
    nb = bp // bm
    return pl.BlockSpec((bm, EMBED_DIM), lambda i, _k=k, _nb=nb: (_k * _nb + i, 0))


@functools.partial(jax.jit, static_argnames=("bm", "part"))
def _mlp_part(xkm, w1k, b1, w2, b2, base, part, bm=1024):
    bp = xkm.shape[0] // CTX
    nb = bp // bm
    off = part * nb
    extra_specs = [] if base is None else [pl.BlockSpec(memory_space=pl.ANY)]
    extra_args = [] if base is None else [base]
    body = _mlp_body_nobase if base is None else _mlp_body
    return pl.pallas_call(
        body,
        grid=(nb,),
        in_specs=[_x_spec(k, bm, bp) for k in range(CTX)]
        + [
            pl.BlockSpec((CTX, EMBED_DIM, EMBED_DIM), lambda i: (0, 0, 0)),
            pl.BlockSpec((1, EMBED_DIM), lambda i: (0, 0)),
            pl.BlockSpec((EMBED_DIM, OUTPUTS), lambda i: (0, 0)),
            pl.BlockSpec((1, OUTPUTS), lambda i: (0, 0)),
        ]
        + extra_specs,
        out_specs=pl.BlockSpec((bm, OUTPUTS), lambda i, _o=off: (_o + i, 0)),
        out_shape=jax.ShapeDtypeStruct((BATCH, OUTPUTS), jnp.float32),
        input_output_aliases={} if base is None else {CTX + 4: 0},
    )(*([xkm] * CTX), w1k, b1, w2, b2, *extra_args)


NPARTS = 2


def kernel(inputs, emb_table, W1, b1, W2, b2):
    bp = BATCH // NPARTS
    iters_p = ITERS // NPARTS
    nbuf = min(NBUF, iters_p)
    while iters_p % nbuf:
        nbuf -= 1
    idx4 = (
        inputs.astype(jnp.int32)
        .T.reshape(CTX, NPARTS, bp)
        .transpose(1, 0, 2)
        .reshape(NPARTS, NW, iters_p, CHUNK)
    )
    w1k = W1.reshape(CTX, EMBED_DIM, EMBED_DIM)
    b1r = b1.reshape(1, EMBED_DIM)
    b2r = b2.reshape(1, OUTPUTS)
    out = None
    for p in range(NPARTS):
        xkm = _sc_gather(idx4[p], emb_table, nbuf=nbuf)
        out = _mlp_part(xkm, w1k, b1r, W2, b2r, out, part=p)
    return out


# speedup vs baseline: 1.0848x; 1.0011x over previous
import functools

import jax
import jax.numpy as jnp
from jax import lax
from jax.experimental import pallas as pl
from jax.experimental.pallas import tpu as pltpu
from jax.experimental.pallas import tpu_sc as plsc

BATCH = 16384
CTX = 6
EMBED_DIM = 128
OUTPUTS = 91

NC = 2
NS = 16
NW = NC * NS

N_IDX = BATCH * CTX
PER_W = N_IDX // NW
CHUNK = 128
ITERS = PER_W // CHUNK
NBUF = 4


def _sc_gather_body(iters, nbuf, idx_hbm, table_hbm, out_hbm, idx_v, bufs, gsem, ssem):
    per_w = iters * CHUNK
    c = lax.axis_index("c")
    s = lax.axis_index("s")
    wid = s * NC + c
    base = wid * per_w
    pltpu.sync_copy(idx_hbm.at[wid], idx_v)

    def start_gather(b, j):
        pltpu.async_copy(table_hbm.at[idx_v.at[j]], bufs.at[b], gsem.at[b])

    def start_store(b, j):
        pltpu.async_copy(
            bufs.at[b], out_hbm.at[pl.ds(base + j * CHUNK, CHUNK)], ssem.at[b]
        )

    def wait_gather(b):
        pltpu.make_async_copy(
            table_hbm.at[idx_v.at[0]], bufs.at[b], gsem.at[b]
        ).wait()

    def wait_store(b):
        pltpu.make_async_copy(
            bufs.at[b], out_hbm.at[pl.ds(base, CHUNK)], ssem.at[b]
        ).wait()

    for b in range(nbuf):
        start_gather(b, b)

    def outer(t, carry):
        j0 = t * nbuf
        for b in range(nbuf):
            wait_gather(b)
            start_store(b, j0 + b)
        for b in range(nbuf):
            wait_store(b)
            start_gather(b, j0 + nbuf + b)
        return carry

    lax.fori_loop(0, iters // nbuf - 1, outer, 0)

    j0 = iters - nbuf
    for b in range(nbuf):
        wait_gather(b)
        start_store(b, j0 + b)
    for b in range(nbuf):
        wait_store(b)


def _sc_gather(idx3, table, nbuf=NBUF):
    nw, iters, chunk = idx3.shape
    mesh = plsc.VectorSubcoreMesh(core_axis_name="c", subcore_axis_name="s")
    return pl.kernel(
        functools.partial(_sc_gather_body, iters, nbuf),
        mesh=mesh,
        out_type=jax.ShapeDtypeStruct((nw * iters * chunk, EMBED_DIM), jnp.float32),
        scratch_types=[
            pltpu.VMEM((iters, CHUNK), jnp.int32),
            pltpu.VMEM((nbuf, CHUNK, EMBED_DIM), jnp.float32),
            pltpu.SemaphoreType.DMA((nbuf,)),
            pltpu.SemaphoreType.DMA((nbuf,)),
        ],
    )(idx3, table)


def _mlp_compute(x_refs, w1_ref, b1_ref, w2_ref, b2_ref, o_ref):
    h = jnp.dot(x_refs[0][...], w1_ref[0], preferred_element_type=jnp.float32)
    for k in range(1, CTX):
        h += jnp.dot(x_refs[k][...], w1_ref[k], preferred_element_type=jnp.float32)
    h = jnp.maximum(h + b1_ref[...], 0.0)
    o_ref[...] = (
        jnp.dot(h, w2_ref[...], preferred_element_type=jnp.float32) + b2_ref[...]
    )


def _mlp_body(*refs):
    _mlp_compute(refs[:CTX], refs[CTX], refs[CTX + 1], refs[CTX + 2],
                 refs[CTX + 3], refs[CTX + 5])


def _mlp_body_nobase(*refs):
    _mlp_compute(refs[:CTX], refs[CTX], refs[CTX + 1], refs[CTX + 2],
                 refs[CTX + 3], refs[CTX + 4])


def _x_spec(k, bm, bp):
    nb = bp // bm
    return pl.BlockSpec((bm, EMBED_DIM), lambda i, _k=k, _nb=nb: (_k * _nb + i, 0))


def _mlp_part(xkm, w1k, b1, w2, b2, base, part, bm=1024):
    bp = xkm.shape[0] // CTX
    nb = bp // bm
    off = part * nb
    extra_specs = [] if base is None else [pl.BlockSpec(memory_space=pl.ANY)]
    extra_args = [] if base is None else [base]
    body = _mlp_body_nobase if base is None else _mlp_body
    return pl.pallas_call(
        body,
        grid=(nb,),
        in_specs=[_x_spec(k, bm, bp) for k in range(CTX)]
        + [
            pl.BlockSpec((CTX, EMBED_DIM, EMBED_DIM), lambda i: (0, 0, 0)),
            pl.BlockSpec((1, EMBED_DIM), lambda i: (0, 0)),
            pl.BlockSpec((EMBED_DIM, OUTPUTS), lambda i: (0, 0)),
            pl.BlockSpec((1, OUTPUTS), lambda i: (0, 0)),
        ]
        + extra_specs,
        out_specs=pl.BlockSpec((bm, OUTPUTS), lambda i, _o=off: (_o + i, 0)),
        out_shape=jax.ShapeDtypeStruct((BATCH, OUTPUTS), jnp.float32),
        input_output_aliases={} if base is None else {CTX + 4: 0},
    )(*([xkm] * CTX), w1k, b1, w2, b2, *extra_args)


NPARTS = 2


def kernel(inputs, emb_table, W1, b1, W2, b2):
    bp = BATCH // NPARTS
    iters_p = ITERS // NPARTS
    nbuf = min(NBUF, iters_p)
    while iters_p % nbuf:
        nbuf -= 1
    idx4 = (
        inputs.astype(jnp.int32)
        .T.reshape(CTX, NPARTS, bp)
        .transpose(1, 0, 2)
        .reshape(NPARTS, NW, iters_p, CHUNK)
    )
    w1k = W1.reshape(CTX, EMBED_DIM, EMBED_DIM)
    b1r = b1.reshape(1, EMBED_DIM)
    b2r = b2.reshape(1, OUTPUTS)
    out = None
    for p in range(NPARTS):
        xkm = _sc_gather(idx4[p], emb_table, nbuf=nbuf)
        out = _mlp_part(xkm, w1k, b1r, W2, b2r, out, part=p)
    return out
